# Initial kernel scaffold; baseline (speedup 1.0000x reference)
import jax
import jax.numpy as jnp
from jax.experimental import pallas as pl


def kernel(x, embeddings):
    raise NotImplementedError("write your pallas kernel here")


# speedup vs baseline: 1.4787x; 1.4787x over previous
import functools

import jax
import jax.numpy as jnp
from jax import lax
from jax.experimental import pallas as pl
from jax.experimental.pallas import tpu as pltpu
from jax.experimental.pallas import tpu_sc as plsc

_N_EMBEDS = 8192
_EMBED_DIM = 32
_BETA = 0.25
_TM = 1024
_TN = 2048
_NT = _N_EMBEDS // _TM
_NC = _N_EMBEDS // _TN

_NW = 32
_BPW = _N_EMBEDS // _NW
_L = 16


def _argmin_kernel(x_ref, emb_ref, idx_ref, part_ref):
    i = pl.program_id(0)
    xt = x_ref[...]
    rownorm = jnp.sum(xt ** 2, axis=1, keepdims=True)
    emb = emb_ref[...]

    m_run = jnp.full((_TM, 1), jnp.inf, jnp.float32)
    i_run = jnp.zeros((_TM, 1), jnp.float32)
    for c in range(_NC):
        emb_c = emb[:, c * _TN:(c + 1) * _TN]
        sim = lax.dot_general(xt, emb_c, (((1,), (0,)), ((), ())),
                              preferred_element_type=jnp.float32)
        enorm = jnp.sum(emb_c ** 2, axis=0, keepdims=True)
        d = rownorm + enorm - 2.0 * sim
        m_c = jnp.min(d, axis=1, keepdims=True)
        iota = lax.broadcasted_iota(jnp.int32, (_TM, _TN), 1).astype(jnp.float32)
        i_loc = jnp.min(jnp.where(d == m_c, iota, jnp.float32(1e9)),
                        axis=1, keepdims=True)
        upd = m_c < m_run
        m_run = jnp.where(upd, m_c, m_run)
        i_run = jnp.where(upd, i_loc + jnp.float32(c * _TN), i_run)

    idx_ref[...] = i_run.astype(jnp.int32)

    dmin_s = jnp.sum(m_run)
    x_s = jnp.sum(xt)

    @pl.when(i == 0)
    def _():
        part_ref[0, 0] = dmin_s
        part_ref[0, 1] = x_s

    @pl.when(i > 0)
    def _():
        part_ref[0, 0] += dmin_s
        part_ref[0, 1] += x_s


def _run_argmin(flat, emb):
    return pl.pallas_call(
        _argmin_kernel,
        grid=(_NT,),
        in_specs=[
            pl.BlockSpec((_TM, _EMBED_DIM), lambda i: (i, 0)),
            pl.BlockSpec((_EMBED_DIM, _N_EMBEDS), lambda i: (0, 0)),
        ],
        out_specs=[
            pl.BlockSpec((_TM, 1), lambda i: (i, 0)),
            pl.BlockSpec((1, 2), lambda i: (0, 0), memory_space=pltpu.SMEM),
        ],
        out_shape=[
            jax.ShapeDtypeStruct((_N_EMBEDS, 1), jnp.int32),
            jax.ShapeDtypeStruct((1, 2), jnp.float32),
        ],
    )(flat, emb)


def _sc_gather_st(table, idx, flat):
    mesh = plsc.VectorSubcoreMesh(core_axis_name="c", subcore_axis_name="s")

    @functools.partial(
        pl.kernel, mesh=mesh,
        out_type=[
            jax.ShapeDtypeStruct((_N_EMBEDS, _EMBED_DIM), jnp.float32),
            jax.ShapeDtypeStruct((_NW, _L), jnp.float32),
        ],
        compiler_params=pltpu.CompilerParams(use_tc_tiling_on_sc=False),
        scratch_types=[
            pltpu.VMEM((_BPW,), jnp.int32),
            pltpu.VMEM((_BPW, _EMBED_DIM), jnp.float32),
            pltpu.VMEM((_BPW, _EMBED_DIM), jnp.float32),
            pltpu.VMEM((_L,), jnp.float32),
            pltpu.SemaphoreType.DMA,
        ],
    )
    def k(table_hbm, idx_hbm, x_hbm, out_hbm, qsum_hbm,
          idx_v, rows_v, x_v, acc_v, sem):
        wid = lax.axis_index("s") * 2 + lax.axis_index("c")
        base = wid * _BPW
        pltpu.sync_copy(idx_hbm.at[pl.ds(base, _BPW)], idx_v)
        gather = pltpu.async_copy(table_hbm.at[idx_v], rows_v, sem)
        pltpu.sync_copy(x_hbm.at[pl.ds(base, _BPW)], x_v)
        gather.wait()

        acc_v[...] = jnp.zeros((_L,), jnp.float32)

        def body(r, acc):
            for h in range(_EMBED_DIM // _L):
                q = rows_v[r, h * _L:(h + 1) * _L]
                xv = x_v[r, h * _L:(h + 1) * _L]
                rows_v[r, h * _L:(h + 1) * _L] = xv + (q - xv)
                acc = acc + q
            return acc

        acc = lax.fori_loop(0, _BPW, body, acc_v[...])
        acc_v[...] = acc
        pltpu.sync_copy(rows_v, out_hbm.at[pl.ds(base, _BPW)])
        pltpu.sync_copy(acc_v, qsum_hbm.at[wid])

    return k(table, idx, flat)


def kernel(x, embeddings):
    in_shape = x.shape
    flat = x.reshape(-1, _EMBED_DIM)
    idx2, parts = _run_argmin(flat, embeddings)
    out, qsum = _sc_gather_st(embeddings.T, idx2[:, 0], flat)
    n = jnp.float32(flat.shape[0] * _EMBED_DIM)
    commit = _BETA * ((jnp.sum(qsum) - parts[0, 1]) / n) ** 2
    codebook = parts[0, 0] / n
    loss = commit + codebook
    return out.reshape(in_shape), loss


# speedup vs baseline: 1.5266x; 1.0324x over previous
import functools

import jax
import jax.numpy as jnp
from jax import lax
from jax.experimental import pallas as pl
from jax.experimental.pallas import tpu as pltpu
from jax.experimental.pallas import tpu_sc as plsc

_N_EMBEDS = 8192
_EMBED_DIM = 32
_BETA = 0.25
_TM = 2048
_TN = 2048
_NT = _N_EMBEDS // _TM
_NC = _N_EMBEDS // _TN

_NW = 32
_BPW = _N_EMBEDS // _NW
_L = 16


def _argmin_kernel(x_ref, emb_ref, idx_ref, part_ref):
    i = pl.program_id(0)
    xt = x_ref[...]
    rownorm = jnp.sum(xt ** 2, axis=1, keepdims=True)
    emb = emb_ref[...]

    m_run = jnp.full((_TM, 1), jnp.inf, jnp.float32)
    i_run = jnp.zeros((_TM, 1), jnp.float32)
    for c in range(_NC):
        emb_c = emb[:, c * _TN:(c + 1) * _TN]
        sim = lax.dot_general(xt, emb_c, (((1,), (0,)), ((), ())),
                              preferred_element_type=jnp.float32)
        enorm = jnp.sum(emb_c ** 2, axis=0, keepdims=True)
        d = rownorm + enorm - 2.0 * sim
        m_c = jnp.min(d, axis=1, keepdims=True)
        iota = lax.broadcasted_iota(jnp.int32, (_TM, _TN), 1).astype(jnp.float32)
        i_loc = jnp.min(jnp.where(d == m_c, iota, jnp.float32(1e9)),
                        axis=1, keepdims=True)
        upd = m_c < m_run
        m_run = jnp.where(upd, m_c, m_run)
        i_run = jnp.where(upd, i_loc + jnp.float32(c * _TN), i_run)

    idx_ref[...] = i_run.astype(jnp.int32)

    dmin_s = jnp.sum(m_run)
    x_s = jnp.sum(xt)

    @pl.when(i == 0)
    def _():
        part_ref[0, 0] = dmin_s
        part_ref[0, 1] = x_s

    @pl.when(i > 0)
    def _():
        part_ref[0, 0] += dmin_s
        part_ref[0, 1] += x_s


def _run_argmin(flat, emb):
    return pl.pallas_call(
        _argmin_kernel,
        grid=(_NT,),
        in_specs=[
            pl.BlockSpec((_TM, _EMBED_DIM), lambda i: (i, 0)),
            pl.BlockSpec((_EMBED_DIM, _N_EMBEDS), lambda i: (0, 0)),
        ],
        out_specs=[
            pl.BlockSpec((_TM, 1), lambda i: (i, 0)),
            pl.BlockSpec((1, 2), lambda i: (0, 0), memory_space=pltpu.SMEM),
        ],
        out_shape=[
            jax.ShapeDtypeStruct((_N_EMBEDS, 1), jnp.int32),
            jax.ShapeDtypeStruct((1, 2), jnp.float32),
        ],
    )(flat, emb)


def _sc_gather_st(table, idx, flat):
    mesh = plsc.VectorSubcoreMesh(core_axis_name="c", subcore_axis_name="s")

    @functools.partial(
        pl.kernel, mesh=mesh,
        out_type=[
            jax.ShapeDtypeStruct((_N_EMBEDS, _EMBED_DIM), jnp.float32),
            jax.ShapeDtypeStruct((_NW, _L), jnp.float32),
        ],
        compiler_params=pltpu.CompilerParams(use_tc_tiling_on_sc=False),
        scratch_types=[
            pltpu.VMEM((_BPW,), jnp.int32),
            pltpu.VMEM((_BPW, _EMBED_DIM), jnp.float32),
            pltpu.VMEM((_BPW, _EMBED_DIM), jnp.float32),
            pltpu.VMEM((_L,), jnp.float32),
            pltpu.SemaphoreType.DMA,
        ],
    )
    def k(table_hbm, idx_hbm, x_hbm, out_hbm, qsum_hbm,
          idx_v, rows_v, x_v, acc_v, sem):
        wid = lax.axis_index("s") * 2 + lax.axis_index("c")
        base = wid * _BPW
        pltpu.sync_copy(idx_hbm.at[pl.ds(base, _BPW)], idx_v)
        gather = pltpu.async_copy(table_hbm.at[idx_v], rows_v, sem)
        pltpu.sync_copy(x_hbm.at[pl.ds(base, _BPW)], x_v)
        gather.wait()

        acc_v[...] = jnp.zeros((_L,), jnp.float32)

        def body(r, acc):
            for h in range(_EMBED_DIM // _L):
                q = rows_v[r, h * _L:(h + 1) * _L]
                xv = x_v[r, h * _L:(h + 1) * _L]
                rows_v[r, h * _L:(h + 1) * _L] = xv + (q - xv)
                acc = acc + q
            return acc

        acc = lax.fori_loop(0, _BPW, body, acc_v[...])
        acc_v[...] = acc
        pltpu.sync_copy(rows_v, out_hbm.at[pl.ds(base, _BPW)])
        pltpu.sync_copy(acc_v, qsum_hbm.at[wid])

    return k(table, idx, flat)


def kernel(x, embeddings):
    in_shape = x.shape
    flat = x.reshape(-1, _EMBED_DIM)
    idx2, parts = _run_argmin(flat, embeddings)
    out, qsum = _sc_gather_st(embeddings.T, idx2[:, 0], flat)
    n = jnp.float32(flat.shape[0] * _EMBED_DIM)
    commit = _BETA * ((jnp.sum(qsum) - parts[0, 1]) / n) ** 2
    codebook = parts[0, 0] / n
    loss = commit + codebook
    return out.reshape(in_shape), loss
